# baseline (device time: 35383 ns/iter reference)
import jax
import jax.numpy as jnp
from jax import lax
from jax.experimental import pallas as pl
from jax.experimental.pallas import tpu as pltpu

N_DEV = 4
K = 16

_IDX_MASK = -4096
_SIGN_FIX = 0x7FFFFFFF
_SENTINEL = -(2**31)


def _pack(x):
    b = lax.bitcast_convert_type(x, jnp.int32)
    key = jnp.where(b >= 0, b, b ^ _SIGN_FIX)
    col = lax.broadcasted_iota(jnp.int32, x.shape, 1)
    return (key & _IDX_MASK) | col


def _unpack(p):
    key = p & _IDX_MASK
    b = jnp.where(key >= 0, key, key ^ _SIGN_FIX)
    return lax.bitcast_convert_type(b, jnp.float32)


def _bitonic_sort_desc(x):
    m, n = x.shape
    lane = lax.broadcasted_iota(jnp.int32, (m, n), 1)
    k = 2
    while k <= n:
        j = k // 2
        while j >= 1:
            up = pltpu.roll(x, n - j, 1)
            down = pltpu.roll(x, j, 1)
            low = (lane & j) == 0
            pv = jnp.where(low, up, down)
            keep_max = low == ((lane & k) == 0)
            x = jnp.where(keep_max, jnp.maximum(x, pv), jnp.minimum(x, pv))
            j //= 2
        k *= 2
    return x


def _local_candidates(p):
    m, n = p.shape
    g = 32
    w = n // g
    t = [jnp.full((m, w), _SENTINEL, jnp.int32) for _ in range(4)]
    for j in range(g):
        x0 = p[:, j * w:(j + 1) * w]
        n0 = jnp.maximum(t[0], x0)
        x1 = jnp.minimum(t[0], x0)
        n1 = jnp.maximum(t[1], x1)
        x2 = jnp.minimum(t[1], x1)
        n2 = jnp.maximum(t[2], x2)
        x3 = jnp.minimum(t[2], x2)
        n3 = jnp.maximum(t[3], x3)
        t = [n0, n1, n2, n3]
    s0 = _bitonic_sort_desc(t[0])
    win = t[0] >= s0[:, 15:16]
    pool = [s0[:, :16]]
    for d, cnt in ((1, 8), (2, 4), (3, 4)):
        sd = _bitonic_sort_desc(jnp.where(win, t[d], _SENTINEL))
        pool.append(sd[:, :cnt])
    return jnp.concatenate(pool, axis=1)


def kernel(x):
    m, n_per = x.shape

    def body(x_ref, out_ref, comm_ref, send_sems, recv_sems):
        my_pos = lax.axis_index("i")

        barrier_sem = pltpu.get_barrier_semaphore()
        for o in range(1, N_DEV):
            pl.semaphore_signal(
                barrier_sem, inc=1,
                device_id=(lax.rem(my_pos + o, N_DEV),),
                device_id_type=pl.DeviceIdType.MESH,
            )
        pl.semaphore_wait(barrier_sem, N_DEV - 1)

        cand = _local_candidates(_pack(x_ref[...]))
        comm_ref[0, :, :] = cand.T

        rdmas = []
        for o in range(1, N_DEV):
            r = pltpu.make_async_remote_copy(
                src_ref=comm_ref.at[0],
                dst_ref=comm_ref.at[o],
                send_sem=send_sems.at[o - 1],
                recv_sem=recv_sems.at[o - 1],
                device_id=(lax.rem(my_pos + o, N_DEV),),
                device_id_type=pl.DeviceIdType.MESH,
            )
            r.start()
            rdmas.append(r)
        for r in rdmas:
            r.wait()

        allc = jnp.concatenate(
            [comm_ref[s, :, :].T for s in range(N_DEV)], axis=1
        )
        col = lax.broadcasted_iota(jnp.int32, allc.shape, 1)
        allc = (allc & _IDX_MASK) | col
        out_ref[...] = _unpack(_bitonic_sort_desc(allc)[:, :K])

    return pl.pallas_call(
        body,
        out_shape=jax.ShapeDtypeStruct((m, K), jnp.float32),
        in_specs=[pl.BlockSpec(memory_space=pltpu.VMEM)],
        out_specs=pl.BlockSpec(memory_space=pltpu.VMEM),
        scratch_shapes=[
            pltpu.VMEM((N_DEV, 2 * K, m), jnp.int32),
            pltpu.SemaphoreType.DMA((N_DEV - 1,)),
            pltpu.SemaphoreType.DMA((N_DEV - 1,)),
        ],
        compiler_params=pltpu.CompilerParams(collective_id=0),
    )(x)


# device time: 18048 ns/iter; 1.9605x vs baseline; 1.9605x over previous
import jax
import jax.numpy as jnp
from jax import lax
from jax.experimental import pallas as pl
from jax.experimental.pallas import tpu as pltpu

N_DEV = 4
K = 16

_IDX_MASK = -4096
_SIGN_FIX = 0x7FFFFFFF
_SENTINEL = -(2**31)


def _pack(x):
    b = lax.bitcast_convert_type(x, jnp.int32)
    key = jnp.where(b >= 0, b, b ^ _SIGN_FIX)
    col = lax.broadcasted_iota(jnp.int32, x.shape, 1)
    return (key & _IDX_MASK) | col


def _unpack(p):
    key = p & _IDX_MASK
    b = jnp.where(key >= 0, key, key ^ _SIGN_FIX)
    return lax.bitcast_convert_type(b, jnp.float32)


def _extract_topk(p, k):
    vals = []
    for _ in range(k):
        v = jnp.max(p, axis=1, keepdims=True)
        vals.append(v)
        p = jnp.where(p == v, _SENTINEL, p)
    return jnp.concatenate(vals, axis=1)


def _staged_pool_t(t, counts=(16, 8, 4, 4)):
    pool = []
    work = t[0]
    ex = None
    for d, cnt in enumerate(counts):
        if d > 0:
            work = jnp.where(ex, t[d], _SENTINEL)
        for _ in range(cnt):
            v = jnp.max(work, axis=0, keepdims=True)
            pool.append(v)
            work = jnp.where(work == v, _SENTINEL, work)
        hit = work == _SENTINEL
        ex = hit if ex is None else ex & hit
    return jnp.concatenate(pool, axis=0)


def _local_candidates(p):
    m, n = p.shape
    g = 32
    w = n // g
    t = [jnp.full((m, w), _SENTINEL, jnp.int32) for _ in range(4)]
    for j in range(g):
        x0 = p[:, j * w:(j + 1) * w]
        n0 = jnp.maximum(t[0], x0)
        x1 = jnp.minimum(t[0], x0)
        n1 = jnp.maximum(t[1], x1)
        x2 = jnp.minimum(t[1], x1)
        n2 = jnp.maximum(t[2], x2)
        x3 = jnp.minimum(t[2], x2)
        n3 = jnp.maximum(t[3], x3)
        t = [n0, n1, n2, n3]
    return _staged_pool_t([jnp.transpose(td) for td in t])


def kernel(x):
    m, n_per = x.shape

    def body(x_ref, out_ref, comm_ref, send_sems, recv_sems):
        my_pos = lax.axis_index("i")

        barrier_sem = pltpu.get_barrier_semaphore()
        for o in range(1, N_DEV):
            pl.semaphore_signal(
                barrier_sem, inc=1,
                device_id=(lax.rem(my_pos + o, N_DEV),),
                device_id_type=pl.DeviceIdType.MESH,
            )
        pl.semaphore_wait(barrier_sem, N_DEV - 1)

        comm_ref[0, :, :] = _local_candidates(_pack(x_ref[...]))

        rdmas = []
        for o in range(1, N_DEV):
            r = pltpu.make_async_remote_copy(
                src_ref=comm_ref.at[0],
                dst_ref=comm_ref.at[o],
                send_sem=send_sems.at[o - 1],
                recv_sem=recv_sems.at[o - 1],
                device_id=(lax.rem(my_pos + o, N_DEV),),
                device_id_type=pl.DeviceIdType.MESH,
            )
            r.start()
            rdmas.append(r)
        for r in rdmas:
            r.wait()

        allc = jnp.concatenate(
            [comm_ref[s, :, :] for s in range(N_DEV)], axis=0
        )
        idx = lax.broadcasted_iota(jnp.int32, allc.shape, 0)
        allc = (allc & _IDX_MASK) | idx
        vals = []
        for _ in range(K):
            v = jnp.max(allc, axis=0, keepdims=True)
            vals.append(v)
            allc = jnp.where(allc == v, _SENTINEL, allc)
        top_t = jnp.concatenate(vals, axis=0)
        out_ref[...] = jnp.transpose(_unpack(top_t))

    return pl.pallas_call(
        body,
        out_shape=jax.ShapeDtypeStruct((m, K), jnp.float32),
        in_specs=[pl.BlockSpec(memory_space=pltpu.VMEM)],
        out_specs=pl.BlockSpec(memory_space=pltpu.VMEM),
        scratch_shapes=[
            pltpu.VMEM((N_DEV, 2 * K, m), jnp.int32),
            pltpu.SemaphoreType.DMA((N_DEV - 1,)),
            pltpu.SemaphoreType.DMA((N_DEV - 1,)),
        ],
        compiler_params=pltpu.CompilerParams(collective_id=0),
    )(x)


# device time: 17227 ns/iter; 2.0539x vs baseline; 1.0477x over previous
import jax
import jax.numpy as jnp
from jax import lax
from jax.experimental import pallas as pl
from jax.experimental.pallas import tpu as pltpu

N_DEV = 4
K = 16

_IDX_MASK = -4096
_SIGN_FIX = 0x7FFFFFFF
_SENTINEL = -(2**31)


def _pack(x):
    b = lax.bitcast_convert_type(x, jnp.int32)
    key = jnp.where(b >= 0, b, b ^ _SIGN_FIX)
    col = lax.broadcasted_iota(jnp.int32, x.shape, 1)
    return (key & _IDX_MASK) | col


def _unpack(p):
    key = p & _IDX_MASK
    b = jnp.where(key >= 0, key, key ^ _SIGN_FIX)
    return lax.bitcast_convert_type(b, jnp.float32)


def _topk_rows_t(a, k):
    vals = []
    for _ in range(k):
        v = jnp.max(a, axis=0, keepdims=True)
        vals.append(v)
        a = jnp.where(a == v, _SENTINEL, a)
    return jnp.concatenate(vals, axis=0)


def _staged_pool_t(t, counts=(16, 8, 4, 4)):
    pool = []
    work = t[0]
    ex = None
    for d, cnt in enumerate(counts):
        if d > 0:
            work = jnp.where(ex, t[d], _SENTINEL)
        for _ in range(cnt):
            v = jnp.max(work, axis=0, keepdims=True)
            pool.append(v)
            work = jnp.where(work == v, _SENTINEL, work)
        hit = work == _SENTINEL
        ex = hit if ex is None else ex & hit
    return jnp.concatenate(pool, axis=0)


def _local_candidates(p):
    m, n = p.shape
    g = 32
    w = n // g
    t = [jnp.full((m, w), _SENTINEL, jnp.int32) for _ in range(4)]
    for j in range(g):
        x0 = p[:, j * w:(j + 1) * w]
        n0 = jnp.maximum(t[0], x0)
        x1 = jnp.minimum(t[0], x0)
        n1 = jnp.maximum(t[1], x1)
        x2 = jnp.minimum(t[1], x1)
        n2 = jnp.maximum(t[2], x2)
        x3 = jnp.minimum(t[2], x2)
        n3 = jnp.maximum(t[3], x3)
        t = [n0, n1, n2, n3]
    return _staged_pool_t([jnp.transpose(td) for td in t])


def kernel(x):
    m, n_per = x.shape

    def body(x_ref, out_ref, comm_ref, send_sems, recv_sems):
        my_pos = lax.axis_index("i")

        barrier_sem = pltpu.get_barrier_semaphore()
        for o in range(1, N_DEV):
            pl.semaphore_signal(
                barrier_sem, inc=1,
                device_id=(lax.rem(my_pos + o, N_DEV),),
                device_id_type=pl.DeviceIdType.MESH,
            )
        pl.semaphore_wait(barrier_sem, N_DEV - 1)

        pool = _local_candidates(_pack(x_ref[...]))
        comm_ref[0, :, :] = _topk_rows_t(pool, K)

        rdmas = []
        for o in range(1, N_DEV):
            r = pltpu.make_async_remote_copy(
                src_ref=comm_ref.at[0],
                dst_ref=comm_ref.at[o],
                send_sem=send_sems.at[o - 1],
                recv_sem=recv_sems.at[o - 1],
                device_id=(lax.rem(my_pos + o, N_DEV),),
                device_id_type=pl.DeviceIdType.MESH,
            )
            r.start()
            rdmas.append(r)
        for r in rdmas:
            r.wait()

        allc = jnp.concatenate(
            [comm_ref[s, :, :] for s in range(N_DEV)], axis=0
        )
        idx = lax.broadcasted_iota(jnp.int32, allc.shape, 0)
        allc = (allc & _IDX_MASK) | idx
        out_ref[...] = jnp.transpose(_unpack(_topk_rows_t(allc, K)))

    return pl.pallas_call(
        body,
        out_shape=jax.ShapeDtypeStruct((m, K), jnp.float32),
        in_specs=[pl.BlockSpec(memory_space=pltpu.VMEM)],
        out_specs=pl.BlockSpec(memory_space=pltpu.VMEM),
        scratch_shapes=[
            pltpu.VMEM((N_DEV, K, m), jnp.int32),
            pltpu.SemaphoreType.DMA((N_DEV - 1,)),
            pltpu.SemaphoreType.DMA((N_DEV - 1,)),
        ],
        compiler_params=pltpu.CompilerParams(collective_id=0),
    )(x)
